# baseline (speedup 1.0000x reference)
import functools

import jax
import jax.numpy as jnp
from jax import lax
from jax.experimental import pallas as pl
from jax.experimental.pallas import tpu as pltpu
from jax.experimental.pallas import tpu_sc as plsc

_LANES = 16
_N_TILES = 32
_CHUNK = 2000


def _pack_rows(lo, hi):
    ulo = lax.bitcast_convert_type(lo, jnp.int32)
    uhi = lax.bitcast_convert_type(hi, jnp.int32)
    r = jnp.int32(0x8000)
    m = jnp.int32(-65536)
    return (lax.shift_right_logical(ulo + r, 16)) | ((uhi + r) & m)


def _mm0_body(x_ref, we_ref, wo_ref, be_ref, bo_ref, o_ref):
    xv = x_ref[...]
    dn = (((0,), (1,)), ((), ()))
    he = lax.dot_general(we_ref[...], xv, dimension_numbers=dn,
                         preferred_element_type=jnp.float32)
    ho = lax.dot_general(wo_ref[...], xv, dimension_numbers=dn,
                         preferred_element_type=jnp.float32)
    o_ref[...] = _pack_rows(he + be_ref[...][:, None],
                            ho + bo_ref[...][:, None])


def _mm1_body(a_ref, we_ref, wo_ref, be_ref, bo_ref, o_ref):
    h = jnp.maximum(a_ref[...], 0.0)
    dn = (((0,), (0,)), ((), ()))
    he = lax.dot_general(we_ref[...], h, dimension_numbers=dn,
                         preferred_element_type=jnp.float32)
    ho = lax.dot_general(wo_ref[...], h, dimension_numbers=dn,
                         preferred_element_type=jnp.float32)
    o_ref[...] = _pack_rows(he + be_ref[...][:, None],
                            ho + bo_ref[...][:, None])


def _lsm_body(n_cls, a_ref, o_ref):
    v = a_ref[...]
    row = lax.broadcasted_iota(jnp.int32, v.shape, 0)
    valid = row < n_cls
    vm = jnp.where(valid, v, -jnp.inf)
    m = jnp.max(vm, axis=0, keepdims=True)
    e = jnp.where(valid, jnp.exp(v - m), 0.0)
    s = jnp.sum(e, axis=0, keepdims=True)
    o_ref[...] = ((v - m) - jnp.log(s)).T[:, :n_cls]


def _mm0(x, we, wo, be, bo):
    hp, npad = we.shape[1], x.shape[0]
    return pl.pallas_call(
        _mm0_body,
        out_shape=jax.ShapeDtypeStruct((hp, npad), jnp.int32),
    )(x, we, wo, be, bo)


def _mm1(a, we, wo, be, bo):
    cp, npad = we.shape[1], a.shape[1]
    return pl.pallas_call(
        _mm1_body,
        out_shape=jax.ShapeDtypeStruct((cp, npad), jnp.int32),
    )(a, we, wo, be, bo)


def _lsm(a, n_cls):
    _, npad = a.shape
    return pl.pallas_call(
        functools.partial(_lsm_body, n_cls),
        out_shape=jax.ShapeDtypeStruct((npad, n_cls), jnp.float32),
    )(a)


def _make_agg(d_feat, npad, epad):
    f_per = d_feat // _N_TILES
    fp = f_per // 2
    c = _CHUNK
    n_pairs = epad // (2 * c)
    groups = c // _LANES

    mesh = plsc.VectorSubcoreMesh(core_axis_name="c", subcore_axis_name="s")

    @functools.partial(
        pl.kernel,
        out_type=jax.ShapeDtypeStruct((d_feat, npad), jnp.float32),
        mesh=mesh,
        compiler_params=pltpu.CompilerParams(needs_layout_passes=False),
        scratch_types=[
            pltpu.VMEM((f_per * npad,), jnp.float32),
            pltpu.VMEM((fp * npad,), jnp.int32),
            pltpu.VMEM((2 * c,), jnp.int32),
            pltpu.VMEM((2 * c,), jnp.int32),
            pltpu.VMEM((2 * c,), jnp.float32),
            pltpu.SemaphoreType.DMA,
            pltpu.SemaphoreType.DMA,
            pltpu.SemaphoreType.DMA,
        ],
    )
    def agg(hP, src, dst, nrm, out, acc, hrows, srcb, dstb, nrmb,
            sem_h, sem_e0, sem_e1):
        wid = lax.axis_index("s") * 2 + lax.axis_index("c")
        f0 = wid * f_per
        p0 = wid * fp
        sems = (sem_e0, sem_e1)

        cp_h = []
        for pc in range(fp):
            cp_h.append(pltpu.async_copy(
                hP.at[p0 + pc], hrows.at[pl.ds(pc * npad, npad)], sem_h))

        def start(g, slot):
            base = g * c
            pltpu.async_copy(src.at[pl.ds(base, c)],
                             srcb.at[pl.ds(slot * c, c)], sems[slot])
            pltpu.async_copy(dst.at[pl.ds(base, c)],
                             dstb.at[pl.ds(slot * c, c)], sems[slot])
            pltpu.async_copy(nrm.at[pl.ds(base, c)],
                             nrmb.at[pl.ds(slot * c, c)], sems[slot])

        def wait(slot):
            pltpu.make_async_copy(src.at[pl.ds(0, c)],
                                  srcb.at[pl.ds(slot * c, c)],
                                  sems[slot]).wait()
            pltpu.make_async_copy(dst.at[pl.ds(0, c)],
                                  dstb.at[pl.ds(slot * c, c)],
                                  sems[slot]).wait()
            pltpu.make_async_copy(nrm.at[pl.ds(0, c)],
                                  nrmb.at[pl.ds(slot * c, c)],
                                  sems[slot]).wait()

        def compute(slot):
            @plsc.parallel_loop(0, groups, unroll=12)
            def gbody(j):
                off = slot * c + j * _LANES
                s16 = srcb[pl.ds(off, _LANES)]
                d16 = dstb[pl.ds(off, _LANES)]
                n16 = nrmb[pl.ds(off, _LANES)]
                mhi = jnp.full((_LANES,), -65536, jnp.int32)
                for pc in range(fp):
                    w = plsc.load_gather(hrows, [s16 + pc * npad])
                    lo = plsc.bitcast(w << 16, jnp.float32)
                    hi = plsc.bitcast(w & mhi, jnp.float32)
                    plsc.addupdate_scatter(
                        acc, [d16 + (2 * pc) * npad], lo * n16)
                    plsc.addupdate_scatter(
                        acc, [d16 + (2 * pc + 1) * npad], hi * n16)

        start(0, 0)

        zeros = jnp.zeros((_LANES,), jnp.float32)

        @plsc.parallel_loop(0, (f_per * npad) // _LANES, unroll=8)
        def zbody(i):
            acc[pl.ds(i * _LANES, _LANES)] = zeros

        for cp in cp_h:
            cp.wait()

        def pair(gp, carry):
            start(2 * gp + 1, 1)
            wait(0)
            compute(0)

            @pl.when(gp + 1 < n_pairs)
            def _():
                start(2 * gp + 2, 0)

            wait(1)
            compute(1)
            return carry

        lax.fori_loop(0, n_pairs, pair, 0)

        for fc in range(f_per):
            pltpu.sync_copy(acc.at[pl.ds(fc * npad, npad)], out.at[f0 + fc])

    return agg


def kernel(x, edge_index, norm, W0, b0, W1, b1):
    n, _ = x.shape
    e = edge_index.shape[1]
    hid = W0.shape[1]
    n_cls = W1.shape[1]

    npad = -(-n // 128) * 128
    epad = -(-e // (2 * _CHUNK)) * (2 * _CHUNK)
    cpad = max(_N_TILES, -(-n_cls // _N_TILES) * _N_TILES)

    if npad == n:
        xp = x.astype(jnp.float32)
    else:
        xp = jnp.zeros((npad, x.shape[1]), jnp.float32).at[:n].set(x)
    ei = edge_index.astype(jnp.int32)
    if epad == e:
        src, dst, nrm = ei[0], ei[1], norm.astype(jnp.float32)
    else:
        src = jnp.zeros((epad,), jnp.int32).at[:e].set(ei[0])
        dst = jnp.zeros((epad,), jnp.int32).at[:e].set(ei[1])
        nrm = jnp.zeros((epad,), jnp.float32).at[:e].set(norm)
    w1p = jnp.zeros((hid, cpad), jnp.float32).at[:, :n_cls].set(W1)
    b1p = jnp.zeros((cpad,), jnp.float32).at[:n_cls].set(b1)

    h0P = _mm0(xp, W0[:, 0::2], W0[:, 1::2], b0[0::2], b0[1::2])
    a0T = _make_agg(hid, npad, epad)(h0P, src, dst, nrm)
    h1P = _mm1(a0T, w1p[:, 0::2], w1p[:, 1::2], b1p[0::2], b1p[1::2])
    a1T = _make_agg(cpad, npad, epad)(h1P, src, dst, nrm)
    outp = _lsm(a1T, n_cls)
    return outp[:n] if npad != n else outp


# speedup vs baseline: 1.0040x; 1.0040x over previous
import functools

import jax
import jax.numpy as jnp
from jax import lax
from jax.experimental import pallas as pl
from jax.experimental.pallas import tpu as pltpu
from jax.experimental.pallas import tpu_sc as plsc

_LANES = 16
_N_TILES = 32
_CHUNK = 2000


def _pack_rows(lo, hi):
    ulo = lax.bitcast_convert_type(lo, jnp.int32)
    uhi = lax.bitcast_convert_type(hi, jnp.int32)
    r = jnp.int32(0x8000)
    m = jnp.int32(-65536)
    return (lax.shift_right_logical(ulo + r, 16)) | ((uhi + r) & m)


def _mm0_body(x_ref, we_ref, wo_ref, be_ref, bo_ref, o_ref):
    xv = x_ref[...]
    dn = (((0,), (1,)), ((), ()))
    he = lax.dot_general(we_ref[...], xv, dimension_numbers=dn,
                         preferred_element_type=jnp.float32)
    ho = lax.dot_general(wo_ref[...], xv, dimension_numbers=dn,
                         preferred_element_type=jnp.float32)
    o_ref[...] = _pack_rows(he + be_ref[...][:, None],
                            ho + bo_ref[...][:, None])


def _mm1_body(a_ref, we_ref, wo_ref, be_ref, bo_ref, o_ref):
    h = jnp.maximum(a_ref[...], 0.0)
    dn = (((0,), (0,)), ((), ()))
    he = lax.dot_general(we_ref[...], h, dimension_numbers=dn,
                         preferred_element_type=jnp.float32)
    ho = lax.dot_general(wo_ref[...], h, dimension_numbers=dn,
                         preferred_element_type=jnp.float32)
    o_ref[...] = _pack_rows(he + be_ref[...][:, None],
                            ho + bo_ref[...][:, None])


def _lsm_body(n_cls, a_ref, o_ref):
    v = a_ref[...]
    row = lax.broadcasted_iota(jnp.int32, v.shape, 0)
    valid = row < n_cls
    vm = jnp.where(valid, v, -jnp.inf)
    m = jnp.max(vm, axis=0, keepdims=True)
    e = jnp.where(valid, jnp.exp(v - m), 0.0)
    s = jnp.sum(e, axis=0, keepdims=True)
    o_ref[...] = ((v - m) - jnp.log(s)).T[:, :n_cls]


def _mm0(x, we, wo, be, bo):
    hp, npad = we.shape[1], x.shape[0]
    return pl.pallas_call(
        _mm0_body,
        out_shape=jax.ShapeDtypeStruct((hp, npad), jnp.int32),
    )(x, we, wo, be, bo)


def _mm1(a, we, wo, be, bo):
    cp, npad = we.shape[1], a.shape[1]
    return pl.pallas_call(
        _mm1_body,
        out_shape=jax.ShapeDtypeStruct((cp, npad), jnp.int32),
    )(a, we, wo, be, bo)


def _lsm(a, n_cls):
    _, npad = a.shape
    return pl.pallas_call(
        functools.partial(_lsm_body, n_cls),
        out_shape=jax.ShapeDtypeStruct((npad, n_cls), jnp.float32),
    )(a)


def _make_agg(d_feat, npad, epad):
    f_per = d_feat // _N_TILES
    fp = f_per // 2
    c = _CHUNK
    n_pairs = epad // (2 * c)
    groups = c // _LANES

    mesh = plsc.VectorSubcoreMesh(core_axis_name="c", subcore_axis_name="s")

    @functools.partial(
        pl.kernel,
        out_type=jax.ShapeDtypeStruct((d_feat, npad), jnp.float32),
        mesh=mesh,
        compiler_params=pltpu.CompilerParams(needs_layout_passes=False),
        scratch_types=[
            pltpu.VMEM((f_per * npad,), jnp.float32),
            pltpu.VMEM((fp * npad,), jnp.int32),
            pltpu.VMEM((2 * c,), jnp.int32),
            pltpu.VMEM((2 * c,), jnp.int32),
            pltpu.VMEM((2 * c,), jnp.float32),
            pltpu.SemaphoreType.DMA,
            pltpu.SemaphoreType.DMA,
            pltpu.SemaphoreType.DMA,
        ],
    )
    def agg(hP, src, dst, nrm, out, acc, hrows, srcb, dstb, nrmb,
            sem_h, sem_e0, sem_e1):
        wid = lax.axis_index("s") * 2 + lax.axis_index("c")
        f0 = wid * f_per
        p0 = wid * fp
        sems = (sem_e0, sem_e1)

        cp_h = []
        for pc in range(fp):
            cp_h.append(pltpu.async_copy(
                hP.at[p0 + pc], hrows.at[pl.ds(pc * npad, npad)], sem_h))

        def start(g, slot):
            base = g * c
            pltpu.async_copy(src.at[pl.ds(base, c)],
                             srcb.at[pl.ds(slot * c, c)], sems[slot])
            pltpu.async_copy(dst.at[pl.ds(base, c)],
                             dstb.at[pl.ds(slot * c, c)], sems[slot])
            pltpu.async_copy(nrm.at[pl.ds(base, c)],
                             nrmb.at[pl.ds(slot * c, c)], sems[slot])

        def wait(slot):
            pltpu.make_async_copy(src.at[pl.ds(0, c)],
                                  srcb.at[pl.ds(slot * c, c)],
                                  sems[slot]).wait()
            pltpu.make_async_copy(dst.at[pl.ds(0, c)],
                                  dstb.at[pl.ds(slot * c, c)],
                                  sems[slot]).wait()
            pltpu.make_async_copy(nrm.at[pl.ds(0, c)],
                                  nrmb.at[pl.ds(slot * c, c)],
                                  sems[slot]).wait()

        def compute(slot):
            @plsc.parallel_loop(0, groups, unroll=8)
            def gbody(j):
                off = slot * c + j * _LANES
                s16 = srcb[pl.ds(off, _LANES)]
                d16 = dstb[pl.ds(off, _LANES)]
                n16 = nrmb[pl.ds(off, _LANES)]
                mhi = jnp.full((_LANES,), -65536, jnp.int32)
                for pc in range(fp):
                    w = plsc.load_gather(hrows, [s16 + pc * npad])
                    lo = plsc.bitcast(w << 16, jnp.float32)
                    hi = plsc.bitcast(w & mhi, jnp.float32)
                    plsc.addupdate_scatter(
                        acc, [d16 + (2 * pc) * npad], lo * n16)
                    plsc.addupdate_scatter(
                        acc, [d16 + (2 * pc + 1) * npad], hi * n16)

        start(0, 0)

        zeros = jnp.zeros((_LANES,), jnp.float32)

        @plsc.parallel_loop(0, (f_per * npad) // _LANES, unroll=8)
        def zbody(i):
            acc[pl.ds(i * _LANES, _LANES)] = zeros

        for cp in cp_h:
            cp.wait()

        def pair(gp, carry):
            start(2 * gp + 1, 1)
            wait(0)
            compute(0)

            @pl.when(gp + 1 < n_pairs)
            def _():
                start(2 * gp + 2, 0)

            wait(1)
            compute(1)
            return carry

        lax.fori_loop(0, n_pairs, pair, 0)

        for fc in range(f_per):
            pltpu.sync_copy(acc.at[pl.ds(fc * npad, npad)], out.at[f0 + fc])

    return agg


def kernel(x, edge_index, norm, W0, b0, W1, b1):
    n, _ = x.shape
    e = edge_index.shape[1]
    hid = W0.shape[1]
    n_cls = W1.shape[1]

    npad = -(-n // 128) * 128
    epad = -(-e // (2 * _CHUNK)) * (2 * _CHUNK)
    cpad = max(_N_TILES, -(-n_cls // _N_TILES) * _N_TILES)

    if npad == n:
        xp = x.astype(jnp.float32)
    else:
        xp = jnp.zeros((npad, x.shape[1]), jnp.float32).at[:n].set(x)
    ei = edge_index.astype(jnp.int32)
    if epad == e:
        src, dst, nrm = ei[0], ei[1], norm.astype(jnp.float32)
    else:
        src = jnp.zeros((epad,), jnp.int32).at[:e].set(ei[0])
        dst = jnp.zeros((epad,), jnp.int32).at[:e].set(ei[1])
        nrm = jnp.zeros((epad,), jnp.float32).at[:e].set(norm)
    w1p = jnp.zeros((hid, cpad), jnp.float32).at[:, :n_cls].set(W1)
    b1p = jnp.zeros((cpad,), jnp.float32).at[:n_cls].set(b1)

    h0P = _mm0(xp, W0[:, 0::2], W0[:, 1::2], b0[0::2], b0[1::2])
    a0T = _make_agg(hid, npad, epad)(h0P, src, dst, nrm)
    h1P = _mm1(a0T, w1p[:, 0::2], w1p[:, 1::2], b1p[0::2], b1p[1::2])
    a1T = _make_agg(cpad, npad, epad)(h1P, src, dst, nrm)
    outp = _lsm(a1T, n_cls)
    return outp[:n] if npad != n else outp
